# baseline (speedup 1.0000x reference)
import functools

import jax
import jax.numpy as jnp
from jax.experimental import pallas as pl
from jax.experimental.pallas import tpu as pltpu


def _emb_body(B, D, word_smem, table_hbm, out_hbm, rows_vmem, sems, wsem):
    NP = 8
    Q = B // NP
    for i in range(B):
        idx = word_smem[i]
        pltpu.make_async_copy(
            table_hbm.at[pl.ds(idx, 1), :],
            rows_vmem.at[pl.ds(i, 1), :],
            sems.at[i // Q],
        ).start(priority=i % 2)
    for p in range(NP):
        pltpu.make_async_copy(
            table_hbm.at[pl.ds(0, Q), :], rows_vmem.at[pl.ds(p * Q, Q), :], sems.at[p]
        ).wait()
        pltpu.make_async_copy(
            rows_vmem.at[pl.ds(p * Q, Q), :], out_hbm.at[pl.ds(p * Q, Q), :], wsem
        ).start()
    pltpu.make_async_copy(rows_vmem, out_hbm, wsem).wait()


def kernel(word, table):
    (B,) = word.shape
    _, D = table.shape

    out = pl.pallas_call(
        functools.partial(_emb_body, B, D),
        in_specs=[
            pl.BlockSpec(memory_space=pltpu.SMEM),
            pl.BlockSpec(memory_space=pl.ANY),
        ],
        out_specs=pl.BlockSpec(memory_space=pl.ANY),
        out_shape=jax.ShapeDtypeStruct((B, D), jnp.float32),
        scratch_shapes=[
            pltpu.VMEM((B, D), jnp.float32),
            pltpu.SemaphoreType.DMA((8,)),
            pltpu.SemaphoreType.DMA,
        ],
    )(word, table)
    return out.reshape(1, 1, -1)


# speedup vs baseline: 1.0103x; 1.0103x over previous
import functools

import jax
import jax.numpy as jnp
from jax.experimental import pallas as pl
from jax.experimental.pallas import tpu as pltpu


def _emb_body(B, D, word_smem, table_hbm, out_hbm, rows_vmem, sems, wsem):
    NP = 4
    Q = B // NP
    for i in range(B):
        idx = word_smem[i]
        pltpu.make_async_copy(
            table_hbm.at[pl.ds(idx, 1), :],
            rows_vmem.at[pl.ds(i, 1), :],
            sems.at[i // Q],
        ).start(priority=i % 2)
    for p in range(NP):
        pltpu.make_async_copy(
            table_hbm.at[pl.ds(0, Q), :], rows_vmem.at[pl.ds(p * Q, Q), :], sems.at[p]
        ).wait()
        pltpu.make_async_copy(
            rows_vmem.at[pl.ds(p * Q, Q), :], out_hbm.at[pl.ds(p * Q, Q), :], wsem
        ).start()
    pltpu.make_async_copy(rows_vmem, out_hbm, wsem).wait()


def kernel(word, table):
    (B,) = word.shape
    _, D = table.shape

    out = pl.pallas_call(
        functools.partial(_emb_body, B, D),
        in_specs=[
            pl.BlockSpec(memory_space=pltpu.SMEM),
            pl.BlockSpec(memory_space=pl.ANY),
        ],
        out_specs=pl.BlockSpec(memory_space=pl.ANY),
        out_shape=jax.ShapeDtypeStruct((B, D), jnp.float32),
        scratch_shapes=[
            pltpu.VMEM((B, D), jnp.float32),
            pltpu.SemaphoreType.DMA((4,)),
            pltpu.SemaphoreType.DMA,
        ],
    )(word, table)
    return out.reshape(1, 1, -1)
